# baseline (speedup 1.0000x reference)
import functools

import jax
import jax.numpy as jnp
from jax import lax
from jax.experimental import pallas as pl
from jax.experimental.pallas import tpu as pltpu
from jax.experimental.pallas import tpu_sc as plsc

_N = 10000
_E = 320000
_F = 128
_R = 16
_NB = 8
_H = 8
_HD = 16
_NEG = 0.2
_EPS = 1e-5

_TILES = 32
_EPT = _E // _TILES
_CH = 80
_NCHUNK = _EPT // _CH
_NPAD = 10112
_RPT = _NPAD // 16
_NBLK = 10
_BN = _N // _NBLK


def _k1_body(wrows_ref, wcomp_ref, out_ref):
    wc = wcomp_ref[...]
    for a in range(_F):
        blk = wrows_ref[a * _NB:(a + 1) * _NB, :]
        out_ref[a * 16:(a + 1) * 16, :] = jnp.dot(
            wc, blk, preferred_element_type=jnp.float32)


def _k1(wrows, w_comp):
    return pl.pallas_call(
        _k1_body,
        out_shape=jax.ShapeDtypeStruct((_F * _R, _F), jnp.float32),
    )(wrows, w_comp)


def _k2_body(x_ref, w_ref, ar_ref, t_ref, sr_ref):
    t = jnp.dot(x_ref[...], w_ref[...], preferred_element_type=jnp.float32)
    t_ref[...] = t
    sr_ref[...] = jnp.dot(x_ref[...], ar_ref[...],
                          preferred_element_type=jnp.float32)


def _k2(x, wfin, armat):
    grid = (_NBLK, _R)
    return pl.pallas_call(
        _k2_body,
        grid=grid,
        in_specs=[
            pl.BlockSpec((_BN, _F), lambda nb, r: (nb, 0)),
            pl.BlockSpec((_F, _F), lambda nb, r: (r, 0)),
            pl.BlockSpec((_F, _H), lambda nb, r: (0, 0)),
        ],
        out_specs=[
            pl.BlockSpec((_BN, _F), lambda nb, r: (r * _NBLK + nb, 0)),
            pl.BlockSpec((_BN, _H), lambda nb, r: (nb, 0)),
        ],
        out_shape=[
            jax.ShapeDtypeStruct((_R * _N, _F), jnp.float32),
            jax.ShapeDtypeStruct((_N, _H), jnp.float32),
        ],
    )(x, wfin, armat)


def _sc_body(t_hbm, sr128_hbm, alf_hbm, fidx_hbm, didx_hbm, d16_hbm,
             zeros_hbm, out_hbm,
             fidx_v, didx_v, d16_v, msg_v, srl_v, al_v, h_sh, sem1, sem2):
    c_id = lax.axis_index("c")
    s_id = lax.axis_index("s")
    wid = c_id * 16 + s_id
    pltpu.sync_copy(alf_hbm, al_v)
    pltpu.sync_copy(zeros_hbm.at[pl.ds(s_id * _RPT, _RPT)],
                    h_sh.at[pl.ds(s_id * _RPT, _RPT)])
    plsc.subcore_barrier()
    rows0 = lax.iota(jnp.int32, 16)

    def group(g, msg_ready):
        rows = rows0 + g * 16
        didx_lane = plsc.load_gather(didx_v, [rows])
        sls = []
        for h in range(_H):
            alc = al_v[pl.ds(h * _HD, _HD)]
            acc = None
            for j in range(_HD):
                f = h * _HD + j
                col = plsc.load_gather(msg_v, [rows, jnp.full((16,), f, jnp.int32)])
                term = col * alc[j]
                acc = term if acc is None else acc + term
            sls.append(acc)
        sub = (didx_lane & 15) * _H
        srs = [plsc.load_gather(srl_v, [rows, sub + h]) for h in range(_H)]
        s = [a + b for a, b in zip(sls, srs)]
        s = [jnp.where(v >= 0.0, v, v * _NEG) for v in s]
        m = functools.reduce(jnp.maximum, s)
        e = [jnp.exp(v - m) for v in s]
        d = functools.reduce(lambda a, b: a + b, e)
        att = [v / d for v in e]
        for f in range(_F):
            cf = jnp.full((16,), f, jnp.int32)
            col = plsc.load_gather(msg_v, [rows, cf])
            plsc.store_scatter(msg_v, [rows, cf], col * att[f // _HD])
        return msg_ready

    def chunk(c, carry):
        base = wid * _EPT + c * _CH
        pltpu.sync_copy(fidx_hbm.at[pl.ds(base, _CH)], fidx_v)
        pltpu.sync_copy(didx_hbm.at[pl.ds(base, _CH)], didx_v)
        pltpu.sync_copy(d16_hbm.at[pl.ds(base, _CH)], d16_v)
        cp1 = pltpu.async_copy(t_hbm.at[fidx_v], msg_v, sem1)
        cp2 = pltpu.async_copy(sr128_hbm.at[d16_v], srl_v, sem2)
        cp1.wait()
        cp2.wait()
        lax.fori_loop(0, _CH // 16, group, 0)
        pltpu.sync_copy(msg_v, h_sh.at[didx_v], add=True)
        return carry

    lax.fori_loop(0, _NCHUNK, chunk, 0)
    plsc.subcore_barrier()
    pltpu.sync_copy(h_sh.at[pl.ds(s_id * _RPT, _RPT)],
                    out_hbm.at[pl.ds(c_id * _NPAD + s_id * _RPT, _RPT)])


def _ksc(t, sr128, al_flat, fidx, didx, d16, zeros):
    mesh = plsc.VectorSubcoreMesh(core_axis_name="c", subcore_axis_name="s")
    f = pl.kernel(
        _sc_body,
        out_type=jax.ShapeDtypeStruct((2 * _NPAD, _F), jnp.float32),
        mesh=mesh,
        compiler_params=pltpu.CompilerParams(needs_layout_passes=False),
        scratch_types=[
            pltpu.VMEM((_CH,), jnp.int32),
            pltpu.VMEM((_CH,), jnp.int32),
            pltpu.VMEM((_CH,), jnp.int32),
            pltpu.VMEM((_CH, _F), jnp.float32),
            pltpu.VMEM((_CH, _F), jnp.float32),
            pltpu.VMEM((_F,), jnp.float32),
            pltpu.VMEM_SHARED((_NPAD, _F), jnp.float32),
            pltpu.SemaphoreType.DMA,
            pltpu.SemaphoreType.DMA,
        ],
    )
    return f(t, sr128, al_flat, fidx, didx, d16, zeros)


def _k4a_body(h0_ref, h1_ref, hsum_ref, s_ref, s2_ref):
    hb = h0_ref[0] + h1_ref[0]
    hsum_ref[...] = hb
    s = jnp.sum(hb, axis=0, keepdims=True)
    s2 = jnp.sum(hb * hb, axis=0, keepdims=True)

    @pl.when(pl.program_id(0) == 0)
    def _():
        s_ref[...] = s
        s2_ref[...] = s2

    @pl.when(pl.program_id(0) != 0)
    def _():
        s_ref[...] += s
        s2_ref[...] += s2


def _k4a(hp2):
    hp3 = hp2.reshape(2, _NPAD, _F)
    return pl.pallas_call(
        _k4a_body,
        grid=(_NBLK,),
        in_specs=[
            pl.BlockSpec((1, _BN, _F), lambda nb: (0, nb, 0)),
            pl.BlockSpec((1, _BN, _F), lambda nb: (1, nb, 0)),
        ],
        out_specs=[
            pl.BlockSpec((_BN, _F), lambda nb: (nb, 0)),
            pl.BlockSpec((1, _F), lambda nb: (0, 0)),
            pl.BlockSpec((1, _F), lambda nb: (0, 0)),
        ],
        out_shape=[
            jax.ShapeDtypeStruct((_N, _F), jnp.float32),
            jax.ShapeDtypeStruct((1, _F), jnp.float32),
            jax.ShapeDtypeStruct((1, _F), jnp.float32),
        ],
    )(hp3, hp3)


def _k4b_body(h_ref, s_ref, s2_ref, g_ref, b_ref, o_ref):
    mean = s_ref[...] / float(_N)
    var = s2_ref[...] / float(_N) - mean * mean
    inv = lax.rsqrt(var + _EPS)
    o_ref[...] = (h_ref[...] - mean) * (inv * g_ref[...]) + b_ref[...]


def _k4b(hsum, s, s2, gamma, beta):
    return pl.pallas_call(
        _k4b_body,
        grid=(_NBLK,),
        in_specs=[
            pl.BlockSpec((_BN, _F), lambda nb: (nb, 0)),
            pl.BlockSpec((1, _F), lambda nb: (0, 0)),
            pl.BlockSpec((1, _F), lambda nb: (0, 0)),
            pl.BlockSpec((1, _F), lambda nb: (0, 0)),
            pl.BlockSpec((1, _F), lambda nb: (0, 0)),
        ],
        out_specs=pl.BlockSpec((_BN, _F), lambda nb: (nb, 0)),
        out_shape=jax.ShapeDtypeStruct((_N, _F), jnp.float32),
    )(hsum, s, s2, gamma, beta)


def kernel(x, edge_index, rel_type, weight, w_comp, attn_l, attn_r,
           bn_gamma, bn_beta):
    wrows = weight.reshape(_NB * _F, _F)
    wfin = _k1(wrows, w_comp)

    oidx = jnp.arange(_F)
    armat = jnp.zeros((_F, _H), jnp.float32).at[oidx, oidx // _HD].set(
        attn_r.reshape(-1))

    t, sr = _k2(x, wfin, armat)

    src = edge_index[0].astype(jnp.int32)
    dst = edge_index[1].astype(jnp.int32)
    fidx = rel_type.astype(jnp.int32) * _N + src
    zeros = jnp.zeros((_NPAD, _F), jnp.float32)

    d16 = dst // 16
    hp2 = _ksc(t, sr.reshape(_N // 16, _F), attn_l.reshape(_F), fidx, dst,
               d16, zeros)
    hsum, s, s2 = _k4a(hp2)
    return _k4b(hsum, s, s2, bn_gamma.reshape(1, _F), bn_beta.reshape(1, _F))


# speedup vs baseline: 3.3819x; 3.3819x over previous
import functools

import jax
import jax.numpy as jnp
from jax import lax
from jax.experimental import pallas as pl
from jax.experimental.pallas import tpu as pltpu
from jax.experimental.pallas import tpu_sc as plsc

_N = 10000
_E = 320000
_F = 128
_R = 16
_NB = 8
_H = 8
_HD = 16
_NEG = 0.2
_EPS = 1e-5

_TILES = 32
_EPT = _E // _TILES
_CH = 64
_EPTP = 10112
_NCH = _EPTP // _CH
_EPAD = _TILES * _EPTP
_NPAD = 10112
_RPT = _NPAD // 16
_NBLK = 10
_BN = _N // _NBLK


def _k1_body(wrows_ref, wcomp_ref, out_ref):
    wc = wcomp_ref[...]
    for a in range(_F):
        blk = wrows_ref[a * _NB:(a + 1) * _NB, :]
        out_ref[a * 16:(a + 1) * 16, :] = jnp.dot(
            wc, blk, preferred_element_type=jnp.float32)


def _k1(wrows, w_comp):
    return pl.pallas_call(
        _k1_body,
        out_shape=jax.ShapeDtypeStruct((_F * _R, _F), jnp.float32),
    )(wrows, w_comp)


def _k2_body(x_ref, w_ref, al_ref, ar_ref, t_ref, sl_ref, sr_ref):
    t = jnp.dot(x_ref[...], w_ref[...], preferred_element_type=jnp.float32)
    t_ref[...] = t
    sl_ref[...] = jnp.dot(t, al_ref[...], preferred_element_type=jnp.float32)
    sr_ref[...] = jnp.dot(x_ref[...], ar_ref[...],
                          preferred_element_type=jnp.float32)


def _k2(x, wfin, almat, armat):
    grid = (_NBLK, _R)
    return pl.pallas_call(
        _k2_body,
        grid=grid,
        in_specs=[
            pl.BlockSpec((_BN, _F), lambda nb, r: (nb, 0)),
            pl.BlockSpec((_F, _F), lambda nb, r: (r, 0)),
            pl.BlockSpec((_F, _H), lambda nb, r: (0, 0)),
            pl.BlockSpec((_F, _F), lambda nb, r: (0, 0)),
        ],
        out_specs=[
            pl.BlockSpec((_BN, _F), lambda nb, r: (r * _NBLK + nb, 0)),
            pl.BlockSpec((_BN, _H), lambda nb, r: (r * _NBLK + nb, 0)),
            pl.BlockSpec((_BN, _F), lambda nb, r: (nb, 0)),
        ],
        out_shape=[
            jax.ShapeDtypeStruct((_R * _N, _F), jnp.float32),
            jax.ShapeDtypeStruct((_R * _N, _H), jnp.float32),
            jax.ShapeDtypeStruct((_N, _F), jnp.float32),
        ],
    )(x, wfin, almat, armat)


def _sc_body(t_hbm, sl16_hbm, sr_hbm, fidx_hbm, f16_hbm, didx_hbm,
             zeros_hbm, out_hbm,
             fidx_v, f16_v, didx_v, didxs_v, msg_v, scat_v, sll_v, srl_v,
             h_sh, fidx_sem, f16_sem, didx_sem, msg_sem, scat_sem, sll_sem,
             srl_sem):
    c_id = lax.axis_index("c")
    s_id = lax.axis_index("s")
    wid = c_id * 16 + s_id
    pltpu.sync_copy(zeros_hbm.at[pl.ds(s_id * _RPT, _RPT)],
                    h_sh.at[pl.ds(s_id * _RPT, _RPT)])
    plsc.subcore_barrier()
    rows0 = lax.iota(jnp.int32, 16)

    def fire_idx(k, b):
        base = wid * _EPTP + k * _CH
        pltpu.async_copy(fidx_hbm.at[pl.ds(base, _CH)], fidx_v[b], fidx_sem[b])
        pltpu.async_copy(f16_hbm.at[pl.ds(base, _CH)], f16_v[b], f16_sem[b])
        pltpu.async_copy(didx_hbm.at[pl.ds(base, _CH)], didx_v[b], didx_sem[b])

    def wait_idx(b):
        pltpu.make_async_copy(fidx_hbm.at[pl.ds(0, _CH)], fidx_v[b],
                              fidx_sem[b]).wait()
        pltpu.make_async_copy(f16_hbm.at[pl.ds(0, _CH)], f16_v[b],
                              f16_sem[b]).wait()
        pltpu.make_async_copy(didx_hbm.at[pl.ds(0, _CH)], didx_v[b],
                              didx_sem[b]).wait()

    def fire_msg(b):
        pltpu.async_copy(t_hbm.at[fidx_v[b]], msg_v[b], msg_sem[b])

    def fire_srl(b):
        pltpu.async_copy(sl16_hbm.at[f16_v[b]], sll_v, sll_sem)
        pltpu.async_copy(sr_hbm.at[didx_v[b]], srl_v, srl_sem)

    def process(b):
        for g in range(_CH // 16):
            rows = rows0 + g * 16
            fl = plsc.load_gather(fidx_v[b], [rows])
            sub = (fl & 15) * _H
            sls = [plsc.load_gather(sll_v, [rows, sub + h])
                   for h in range(_H)]
            srs = [plsc.load_gather(srl_v,
                                    [rows, jnp.full((16,), h, jnp.int32)])
                   for h in range(_H)]
            s = [a + bb for a, bb in zip(sls, srs)]
            s = [jnp.where(v >= 0.0, v, v * _NEG) for v in s]
            m = functools.reduce(jnp.maximum, s)
            e = [jnp.exp(v - m) for v in s]
            d = functools.reduce(lambda a, bb: a + bb, e)
            att = [v / d for v in e]
            for el in range(16):
                row = g * 16 + el
                for h in range(_H):
                    a_s = att[h][el]
                    mcol = msg_v[b][row, pl.ds(h * _HD, _HD)]
                    scat_v[b][row, pl.ds(h * _HD, _HD)] = mcol * a_s

    def half(k, b):
        @pl.when(k + 1 < _NCH)
        def _():
            wait_idx(1 - b)
            fire_msg(1 - b)

        pltpu.make_async_copy(t_hbm.at[fidx_v[b]], msg_v[b], msg_sem[b]).wait()
        pltpu.make_async_copy(sl16_hbm.at[f16_v[b]], sll_v, sll_sem).wait()
        pltpu.make_async_copy(sr_hbm.at[didx_v[b]], srl_v, srl_sem).wait()

        @pl.when(k >= 2)
        def _():
            pltpu.make_async_copy(scat_v[b], h_sh.at[didxs_v[b]],
                                  scat_sem[b]).wait()

        process(b)

        @pl.when(k + 1 < _NCH)
        def _():
            fire_srl(1 - b)

        for j in range(_CH // 16):
            didxs_v[b][pl.ds(j * 16, 16)] = didx_v[b][pl.ds(j * 16, 16)]
        pltpu.async_copy(scat_v[b], h_sh.at[didxs_v[b]], scat_sem[b], add=True)

        @pl.when(k + 2 < _NCH)
        def _():
            fire_idx(k + 2, b)

    fire_idx(0, 0)
    wait_idx(0)
    fire_msg(0)
    fire_srl(0)
    fire_idx(1, 1)

    def pair(c2, carry):
        half(2 * c2, 0)
        half(2 * c2 + 1, 1)
        return carry

    lax.fori_loop(0, _NCH // 2, pair, 0)
    for b in range(2):
        pltpu.make_async_copy(scat_v[b], h_sh.at[didxs_v[b]],
                              scat_sem[b]).wait()
    plsc.subcore_barrier()
    pltpu.sync_copy(h_sh.at[pl.ds(s_id * _RPT, _RPT)],
                    out_hbm.at[pl.ds(c_id * _NPAD + s_id * _RPT, _RPT)])


def _ksc(t, sl16, sr_rep, fidx, f16, didx, zeros):
    mesh = plsc.VectorSubcoreMesh(core_axis_name="c", subcore_axis_name="s")
    f = pl.kernel(
        _sc_body,
        out_type=jax.ShapeDtypeStruct((2 * _NPAD, _F), jnp.float32),
        mesh=mesh,
        compiler_params=pltpu.CompilerParams(needs_layout_passes=False),
        scratch_types=[
            [pltpu.VMEM((_CH,), jnp.int32) for _ in range(2)],
            [pltpu.VMEM((_CH,), jnp.int32) for _ in range(2)],
            [pltpu.VMEM((_CH,), jnp.int32) for _ in range(2)],
            [pltpu.VMEM((_CH,), jnp.int32) for _ in range(2)],
            [pltpu.VMEM((_CH, _F), jnp.float32) for _ in range(2)],
            [pltpu.VMEM((_CH, _F), jnp.float32) for _ in range(2)],
            pltpu.VMEM((_CH, _F), jnp.float32),
            pltpu.VMEM((_CH, _F), jnp.float32),
            pltpu.VMEM_SHARED((_NPAD, _F), jnp.float32),
            [pltpu.SemaphoreType.DMA for _ in range(2)],
            [pltpu.SemaphoreType.DMA for _ in range(2)],
            [pltpu.SemaphoreType.DMA for _ in range(2)],
            [pltpu.SemaphoreType.DMA for _ in range(2)],
            [pltpu.SemaphoreType.DMA for _ in range(2)],
            pltpu.SemaphoreType.DMA,
            pltpu.SemaphoreType.DMA,
        ],
    )
    return f(t, sl16, sr_rep, fidx, f16, didx, zeros)


def _k4a_body(h0_ref, h1_ref, hsum_ref, s_ref, s2_ref):
    hb = h0_ref[0] + h1_ref[0]
    hsum_ref[...] = hb
    s = jnp.sum(hb, axis=0, keepdims=True)
    s2 = jnp.sum(hb * hb, axis=0, keepdims=True)

    @pl.when(pl.program_id(0) == 0)
    def _():
        s_ref[...] = s
        s2_ref[...] = s2

    @pl.when(pl.program_id(0) != 0)
    def _():
        s_ref[...] += s
        s2_ref[...] += s2


def _k4a(hp2):
    hp3 = hp2.reshape(2, _NPAD, _F)
    return pl.pallas_call(
        _k4a_body,
        grid=(_NBLK,),
        in_specs=[
            pl.BlockSpec((1, _BN, _F), lambda nb: (0, nb, 0)),
            pl.BlockSpec((1, _BN, _F), lambda nb: (1, nb, 0)),
        ],
        out_specs=[
            pl.BlockSpec((_BN, _F), lambda nb: (nb, 0)),
            pl.BlockSpec((1, _F), lambda nb: (0, 0)),
            pl.BlockSpec((1, _F), lambda nb: (0, 0)),
        ],
        out_shape=[
            jax.ShapeDtypeStruct((_N, _F), jnp.float32),
            jax.ShapeDtypeStruct((1, _F), jnp.float32),
            jax.ShapeDtypeStruct((1, _F), jnp.float32),
        ],
    )(hp3, hp3)


def _k4b_body(h_ref, s_ref, s2_ref, g_ref, b_ref, o_ref):
    mean = s_ref[...] / float(_N)
    var = s2_ref[...] / float(_N) - mean * mean
    inv = lax.rsqrt(var + _EPS)
    o_ref[...] = (h_ref[...] - mean) * (inv * g_ref[...]) + b_ref[...]


def _k4b(hsum, s, s2, gamma, beta):
    return pl.pallas_call(
        _k4b_body,
        grid=(_NBLK,),
        in_specs=[
            pl.BlockSpec((_BN, _F), lambda nb: (nb, 0)),
            pl.BlockSpec((1, _F), lambda nb: (0, 0)),
            pl.BlockSpec((1, _F), lambda nb: (0, 0)),
            pl.BlockSpec((1, _F), lambda nb: (0, 0)),
            pl.BlockSpec((1, _F), lambda nb: (0, 0)),
        ],
        out_specs=pl.BlockSpec((_BN, _F), lambda nb: (nb, 0)),
        out_shape=jax.ShapeDtypeStruct((_N, _F), jnp.float32),
    )(hsum, s, s2, gamma, beta)


def kernel(x, edge_index, rel_type, weight, w_comp, attn_l, attn_r,
           bn_gamma, bn_beta):
    wrows = weight.reshape(_NB * _F, _F)
    wfin = _k1(wrows, w_comp)

    oidx = jnp.arange(_F)
    armat_pad = jnp.zeros((_F, _F), jnp.float32).at[oidx, oidx // _HD].set(
        attn_r.reshape(-1))
    almat = jnp.zeros((_F, _H), jnp.float32).at[oidx, oidx // _HD].set(
        attn_l.reshape(-1))

    t, sl, sr = _k2(x, wfin, almat, armat_pad)
    sl16 = sl.reshape(_R * _N // 16, _F)
    sr_rep = jnp.concatenate(
        [sr, jnp.zeros((_NPAD - _N, _F), jnp.float32)], axis=0)

    src_i = edge_index[0].astype(jnp.int32)
    dst = edge_index[1].astype(jnp.int32)
    fidx = rel_type.astype(jnp.int32) * _N + src_i
    npad = _EPAD - _E
    fidx_p = jnp.concatenate([fidx, jnp.zeros((npad,), jnp.int32)])
    didx_p = jnp.concatenate([dst, jnp.full((npad,), _N, jnp.int32)])
    f16_p = fidx_p // 16
    zeros = jnp.zeros((_NPAD, _F), jnp.float32)

    hp2 = _ksc(t, sl16, sr_rep, fidx_p, f16_p, didx_p, zeros)
    hsum, s, s2 = _k4a(hp2)
    return _k4b(hsum, s, s2, bn_gamma.reshape(1, _F), bn_beta.reshape(1, _F))
